# Initial kernel scaffold; baseline (speedup 1.0000x reference)
import functools

import jax
import jax.numpy as jnp
from jax import lax
from jax.experimental import pallas as pl
from jax.experimental.pallas import tpu as pltpu
from jax.experimental.pallas import tpu_sc as plsc

_M = 16
_XMIN = -3.5
_XMAX = 3.5
_NW = 32
_C = 2048
_L = 16


@functools.partial(jax.jit, static_argnums=(1,))
def _hat_spline_sc(xf, T):
    spw = T // _NW
    nchunks = spw // _C

    h = (_XMAX - _XMIN) / (_M - 1)
    inv_h = jnp.float32(1.0 / h)
    inv_d = jnp.float32(1.0 / (h + 1e-8))

    mesh = plsc.VectorSubcoreMesh(core_axis_name="c", subcore_axis_name="s")

    @functools.partial(
        pl.kernel,
        out_type=jax.ShapeDtypeStruct((T * _M,), jnp.float32),
        mesh=mesh,
        scratch_types=[
            pltpu.VMEM((_C,), jnp.float32),
            pltpu.VMEM((_C * _M,), jnp.float32),
        ],
    )
    def sc_kernel(x_hbm, out_hbm, xv, ov):
        wid = lax.axis_index("s") * 2 + lax.axis_index("c")
        base = wid * spw
        lanes = lax.iota(jnp.int32, _L)
        zeros = jnp.zeros((_L,), jnp.float32)

        def chunk_body(ci, carry):
            off = base + ci * _C
            pltpu.sync_copy(x_hbm.at[pl.ds(off, _C)], xv)

            def zbody(k, c):
                ov[pl.ds(k * _L, _L)] = zeros
                return c

            lax.fori_loop(0, _C * _M // _L, zbody, 0)

            def cbody(j, c):
                xr = xv[pl.ds(j * _L, _L)]
                xc = jnp.minimum(jnp.maximum(xr, _XMIN), _XMAX)
                t = (xc - _XMIN) * inv_h
                it = jnp.minimum(t.astype(jnp.int32), _M - 2)
                x0 = it.astype(jnp.float32) * jnp.float32(h) + _XMIN
                w = (xc - x0) * inv_d
                fidx = (j * (_L * _M) + lanes * _M) + it
                plsc.store_scatter(ov, [fidx], 1.0 - w)
                plsc.store_scatter(ov, [fidx + 1], w)
                return c

            lax.fori_loop(0, _C // _L, cbody, 0)
            pltpu.sync_copy(ov, out_hbm.at[pl.ds(off * _M, _C * _M)])
            return carry

        lax.fori_loop(0, nchunks, chunk_body, 0)

    return sc_kernel(xf)


def kernel(x):
    B, N = x.shape
    T = B * N
    out = _hat_spline_sc(x.reshape(-1), T)
    return out.reshape(B, N, _M)


# speedup vs baseline: 11.6458x; 11.6458x over previous
import functools

import jax
import jax.numpy as jnp
from jax import lax
from jax.experimental import pallas as pl
from jax.experimental.pallas import tpu as pltpu
from jax.experimental.pallas import tpu_sc as plsc

_M = 16
_XMIN = -3.5
_XMAX = 3.5
_NW = 32
_C = 2048
_L = 16


@functools.partial(jax.jit, static_argnums=(1,))
def _hat_spline_sc(xf, T):
    spw = T // _NW
    nchunks = spw // _C

    h = (_XMAX - _XMIN) / (_M - 1)
    inv_h = jnp.float32(1.0 / h)
    inv_d = jnp.float32(1.0 / (h + 1e-8))

    mesh = plsc.VectorSubcoreMesh(core_axis_name="c", subcore_axis_name="s")

    @functools.partial(
        pl.kernel,
        out_type=jax.ShapeDtypeStruct((T * _M,), jnp.float32),
        mesh=mesh,
        scratch_types=[
            pltpu.VMEM((_C,), jnp.float32),
            pltpu.VMEM((_C * _M,), jnp.float32),
        ],
        compiler_params=pltpu.CompilerParams(needs_layout_passes=False),
    )
    def sc_kernel(x_hbm, out_hbm, xv, ov):
        wid = lax.axis_index("s") * 2 + lax.axis_index("c")
        base = wid * spw
        lanes = lax.iota(jnp.int32, _L)
        zeros = jnp.zeros((_L,), jnp.float32)

        def chunk_body(ci, carry):
            off = base + ci * _C
            pltpu.sync_copy(x_hbm.at[pl.ds(off, _C)], xv)

            def zbody(k, c):
                ov[pl.ds(k * _L, _L)] = zeros
                return c

            lax.fori_loop(0, _C * _M // _L, zbody, 0)

            def cbody(j, c):
                xr = xv[pl.ds(j * _L, _L)]
                xc = jnp.minimum(jnp.maximum(xr, _XMIN), _XMAX)
                t = (xc - _XMIN) * inv_h
                it = jnp.minimum(t.astype(jnp.int32), _M - 2)
                x0 = it.astype(jnp.float32) * jnp.float32(h) + _XMIN
                w = (xc - x0) * inv_d
                fidx = (j * (_L * _M) + lanes * _M) + it
                plsc.store_scatter(ov, [fidx], 1.0 - w)
                plsc.store_scatter(ov, [fidx + 1], w)
                return c

            lax.fori_loop(0, _C // _L, cbody, 0)
            pltpu.sync_copy(ov, out_hbm.at[pl.ds(off * _M, _C * _M)])
            return carry

        lax.fori_loop(0, nchunks, chunk_body, 0)

    return sc_kernel(xf)


def kernel(x):
    B, N = x.shape
    T = B * N
    out = _hat_spline_sc(x.reshape(-1), T)
    return out.reshape(B, N, _M)


# speedup vs baseline: 16.9897x; 1.4589x over previous
import functools

import jax
import jax.numpy as jnp
from jax import lax
from jax.experimental import pallas as pl
from jax.experimental.pallas import tpu as pltpu
from jax.experimental.pallas import tpu_sc as plsc

_M = 16
_XMIN = -3.5
_XMAX = 3.5
_NW = 32
_C = 2048
_L = 16


@functools.partial(jax.jit, static_argnums=(1,))
def _hat_spline_sc(xf, T):
    spw = T // _NW
    nchunks = spw // _C

    h = (_XMAX - _XMIN) / (_M - 1)
    inv_h = jnp.float32(1.0 / h)
    inv_d = jnp.float32(1.0 / (h + 1e-8))

    mesh = plsc.VectorSubcoreMesh(core_axis_name="c", subcore_axis_name="s")

    @functools.partial(
        pl.kernel,
        out_type=jax.ShapeDtypeStruct((T * _M,), jnp.float32),
        mesh=mesh,
        scratch_types=[
            pltpu.VMEM((_C,), jnp.float32),
            pltpu.VMEM((_C,), jnp.float32),
            pltpu.VMEM((_C * _M,), jnp.float32),
            pltpu.VMEM((_C * _M,), jnp.float32),
            pltpu.VMEM((_C,), jnp.int32),
            pltpu.VMEM((_C,), jnp.int32),
            pltpu.SemaphoreType.DMA,
            pltpu.SemaphoreType.DMA,
            pltpu.SemaphoreType.DMA,
            pltpu.SemaphoreType.DMA,
        ],
        compiler_params=pltpu.CompilerParams(needs_layout_passes=False),
    )
    def sc_kernel(x_hbm, out_hbm, xv0, xv1, ov0, ov1, ib0, ib1,
                  isem0, isem1, osem0, osem1):
        wid = lax.axis_index("s") * 2 + lax.axis_index("c")
        base = wid * spw
        lanes_m = lax.iota(jnp.int32, _L) * _M
        zeros = jnp.zeros((_L,), jnp.float32)
        bufs = ((xv0, ov0, ib0, isem0, osem0), (xv1, ov1, ib1, isem1, osem1))

        pltpu.async_copy(x_hbm.at[pl.ds(base, _C)], xv0, isem0)
        pltpu.async_copy(x_hbm.at[pl.ds(base + _C, _C)], xv1, isem1)

        for _, ov_b, ib_b, _, _ in bufs:

            @plsc.parallel_loop(0, _C * _M // _L, unroll=8)
            def _zero(k, ov_b=ov_b):
                ov_b[pl.ds(k * _L, _L)] = zeros

            @plsc.parallel_loop(0, _C // _L, unroll=4)
            def _init(j, ib_b=ib_b):
                ib_b[pl.ds(j * _L, _L)] = j * (_L * _M) + lanes_m

        @pl.loop(0, nchunks // 2)
        def _chunks(cc):
            for b in range(2):
                xv_b, ov_b, ib_b, isem_b, osem_b = bufs[b]
                ci = cc * 2 + b
                off = base + ci * _C

                @pl.when(cc > 0)
                def _drain(ov_b=ov_b, osem_b=osem_b):
                    pltpu.make_async_copy(
                        ov_b, out_hbm.at[pl.ds(0, _C * _M)], osem_b
                    ).wait()

                pltpu.make_async_copy(
                    x_hbm.at[pl.ds(0, _C)], xv_b, isem_b
                ).wait()

                @plsc.parallel_loop(0, _C // _L, unroll=4)
                def _fused(j, xv_b=xv_b, ov_b=ov_b, ib_b=ib_b):
                    xr = xv_b[pl.ds(j * _L, _L)]
                    fold = ib_b[pl.ds(j * _L, _L)]
                    plsc.store_scatter(ov_b, [fold], zeros)
                    plsc.store_scatter(ov_b, [fold + 1], zeros)
                    xc = jnp.minimum(jnp.maximum(xr, _XMIN), _XMAX)
                    t = (xc - _XMIN) * inv_h
                    it = jnp.minimum(t.astype(jnp.int32), _M - 2)
                    x0 = it.astype(jnp.float32) * jnp.float32(h) + _XMIN
                    w = (xc - x0) * inv_d
                    fidx = (j * (_L * _M) + lanes_m) + it
                    plsc.store_scatter(ov_b, [fidx], 1.0 - w)
                    plsc.store_scatter(ov_b, [fidx + 1], w)
                    ib_b[pl.ds(j * _L, _L)] = fidx

                @pl.when(cc + 1 < nchunks // 2)
                def _prefetch(xv_b=xv_b, isem_b=isem_b, off=off):
                    pltpu.async_copy(
                        x_hbm.at[pl.ds(off + 2 * _C, _C)], xv_b, isem_b
                    )

                pltpu.async_copy(
                    ov_b, out_hbm.at[pl.ds(off * _M, _C * _M)], osem_b
                )

        for _, ov_b, _, _, osem_b in bufs:
            pltpu.make_async_copy(
                ov_b, out_hbm.at[pl.ds(0, _C * _M)], osem_b
            ).wait()

    return sc_kernel(xf)


def kernel(x):
    B, N = x.shape
    T = B * N
    out = _hat_spline_sc(x.reshape(-1), T)
    return out.reshape(B, N, _M)


# speedup vs baseline: 197.5071x; 11.6251x over previous
import functools

import jax
import jax.numpy as jnp
from jax import lax
from jax.experimental import pallas as pl
from jax.experimental.pallas import tpu as pltpu
from jax.experimental.pallas import tpu_sc as plsc

_M = 16
_XMIN = -3.5
_XMAX = 3.5
_NW = 32
_C = 2048
_L = 16


@functools.partial(jax.jit, static_argnums=(1, 2))
def _hat_spline_sc(xp, B, N):
    T = B * N
    spw = T // _NW
    nchunks = spw // _C
    ntiles_w = spw // 1024
    bplane = N * _M
    mplane = bplane // 2

    h = (_XMAX - _XMIN) / (_M - 1)
    inv_h = jnp.float32(1.0 / h)
    inv_d = jnp.float32(1.0 / (h + 1e-8))

    mesh = plsc.VectorSubcoreMesh(core_axis_name="c", subcore_axis_name="s")

    @functools.partial(
        pl.kernel,
        out_type=jax.ShapeDtypeStruct((T * _M,), jnp.float32),
        mesh=mesh,
        scratch_types=[
            pltpu.VMEM((_C,), jnp.float32),
            pltpu.VMEM((_C,), jnp.float32),
            pltpu.VMEM((_C * _M,), jnp.float32),
            pltpu.VMEM((_C * _M,), jnp.float32),
            pltpu.VMEM((_C,), jnp.int32),
            pltpu.VMEM((_C,), jnp.int32),
            pltpu.SemaphoreType.DMA,
            pltpu.SemaphoreType.DMA,
            pltpu.SemaphoreType.DMA,
            pltpu.SemaphoreType.DMA,
        ],
        compiler_params=pltpu.CompilerParams(needs_layout_passes=False),
    )
    def sc_kernel(x_hbm, out_hbm, xv0, xv1, ov0, ov1, ib0, ib1,
                  isem0, isem1, osem0, osem1):
        wid = lax.axis_index("s") * 2 + lax.axis_index("c")
        base = wid * spw
        bt = wid // (_NW // 2)
        ntbase = (wid % (_NW // 2)) * ntiles_w
        lanes = lax.iota(jnp.int32, _L)
        zeros = jnp.zeros((_L,), jnp.float32)
        bufs = ((xv0, ov0, ib0, isem0, osem0), (xv1, ov1, ib1, isem1, osem1))

        pltpu.async_copy(x_hbm.at[pl.ds(base, _C)], xv0, isem0)
        pltpu.async_copy(x_hbm.at[pl.ds(base + _C, _C)], xv1, isem1)

        for _, ov_b, ib_b, _, _ in bufs:

            @plsc.parallel_loop(0, _C * _M // _L, unroll=8)
            def _zero(k, ov_b=ov_b):
                ov_b[pl.ds(k * _L, _L)] = zeros

            @plsc.parallel_loop(0, _C // _L, unroll=4)
            def _init(j, ib_b=ib_b):
                base_j = (
                    ((j >> 3) & 7) * 4096 + (j >> 6) * 1024 + (j & 7) * _L
                )
                ib_b[pl.ds(j * _L, _L)] = base_j + lanes

        @pl.loop(0, nchunks // 2)
        def _chunks(cc):
            for b in range(2):
                xv_b, ov_b, ib_b, isem_b, osem_b = bufs[b]
                ci = cc * 2 + b
                off = base + ci * _C
                nt0 = ntbase + ci * 2

                @pl.when(cc > 0)
                def _drain(ov_b=ov_b, osem_b=osem_b):
                    pltpu.make_async_copy(
                        ov_b, out_hbm.at[pl.ds(0, _C * _M)], osem_b
                    ).wait()

                pltpu.make_async_copy(
                    x_hbm.at[pl.ds(0, _C)], xv_b, isem_b
                ).wait()

                @plsc.parallel_loop(0, _C // _L, unroll=4)
                def _fused(j, xv_b=xv_b, ov_b=ov_b, ib_b=ib_b):
                    xr = xv_b[pl.ds(j * _L, _L)]
                    fold = ib_b[pl.ds(j * _L, _L)]
                    dold = jnp.where(
                        ((fold >> 7) & 7) == 7, 1152, 128
                    ).astype(jnp.int32)
                    plsc.store_scatter(ov_b, [fold], zeros)
                    plsc.store_scatter(ov_b, [fold + dold], zeros)
                    xc = jnp.minimum(jnp.maximum(xr, _XMIN), _XMAX)
                    t = (xc - _XMIN) * inv_h
                    it = jnp.minimum(t.astype(jnp.int32), _M - 2)
                    x0 = it.astype(jnp.float32) * jnp.float32(h) + _XMIN
                    w = (xc - x0) * inv_d
                    base_j = (
                        ((j >> 3) & 7) * 4096 + (j >> 6) * 1024 + (j & 7) * _L
                    )
                    fidx = (
                        (base_j + lanes)
                        + ((it >> 3) << 11)
                        + ((it & 7) << 7)
                    )
                    dnew = jnp.where((it & 7) == 7, 1152, 128).astype(
                        jnp.int32
                    )
                    plsc.store_scatter(ov_b, [fidx], 1.0 - w)
                    plsc.store_scatter(ov_b, [fidx + dnew], w)
                    ib_b[pl.ds(j * _L, _L)] = fidx

                @pl.when(cc + 1 < nchunks // 2)
                def _prefetch(xv_b=xv_b, isem_b=isem_b, off=off):
                    pltpu.async_copy(
                        x_hbm.at[pl.ds(off + 2 * _C, _C)], xv_b, isem_b
                    )

                for bs in range(8):
                    for mt in range(2):
                        dst = (
                            (bt * 8 + bs) * bplane
                            + mt * mplane
                            + nt0 * 1024
                        )
                        pltpu.async_copy(
                            ov_b.at[pl.ds((bs * 2 + mt) * 2048, 2048)],
                            out_hbm.at[pl.ds(dst, 2048)],
                            osem_b,
                        )

        for _, ov_b, _, _, osem_b in bufs:
            pltpu.make_async_copy(
                ov_b, out_hbm.at[pl.ds(0, _C * _M)], osem_b
            ).wait()

    return sc_kernel(xp)


def kernel(x):
    B, N = x.shape
    xp = (
        x.reshape(B // 8, 8, N // 128, 128)
        .transpose(0, 2, 1, 3)
        .reshape(-1)
    )
    flat = _hat_spline_sc(xp, B, N)
    out = (
        flat.reshape(B, 2, N // 128, 8, 128)
        .transpose(0, 2, 4, 1, 3)
        .reshape(B, N, _M)
    )
    return out
